# baseline (speedup 1.0000x reference)
import functools

import jax
import jax.numpy as jnp
from jax import lax
from jax.experimental import pallas as pl
from jax.experimental.pallas import tpu as pltpu
from jax.experimental.pallas import tpu_sc as plsc

B = 16384
H = 64

_info = plsc.get_sparse_core_info()
_NC = _info.num_cores
_NS = _info.num_subcores
_NW = _NC * _NS
_BPW = B // _NW


def _sc_gather_body(emb_p, emb_n, idx_p_hbm, idx_n_hbm, out_p, out_n,
                    idxp_v, idxn_v, rowsp_v, rowsn_v, semp, semn):
    wid = lax.axis_index("s") * _NC + lax.axis_index("c")
    base = wid * _BPW
    pltpu.sync_copy(idx_p_hbm.at[pl.ds(base, _BPW)], idxp_v)
    pltpu.sync_copy(idx_n_hbm.at[pl.ds(base, _BPW)], idxn_v)
    cp = pltpu.async_copy(emb_p.at[idxp_v], rowsp_v, semp)
    cn = pltpu.async_copy(emb_n.at[idxn_v], rowsn_v, semn)
    cp.wait()
    cn.wait()
    pltpu.sync_copy(rowsp_v, out_p.at[pl.ds(base, _BPW)])
    pltpu.sync_copy(rowsn_v, out_n.at[pl.ds(base, _BPW)])


_sc_gather = functools.partial(
    pl.kernel,
    mesh=plsc.VectorSubcoreMesh(core_axis_name="c", subcore_axis_name="s"),
    out_type=[
        jax.ShapeDtypeStruct((B, H), jnp.float32),
        jax.ShapeDtypeStruct((B, H), jnp.float32),
    ],
    scratch_types=[
        pltpu.VMEM((_BPW,), jnp.int32),
        pltpu.VMEM((_BPW,), jnp.int32),
        pltpu.VMEM((_BPW, H), jnp.float32),
        pltpu.VMEM((_BPW, H), jnp.float32),
        pltpu.SemaphoreType.DMA,
        pltpu.SemaphoreType.DMA,
    ],
    compiler_params=pltpu.CompilerParams(use_tc_tiling_on_sc=False),
)(_sc_gather_body)


_BM = 2048


def _mlp_body(p_ref, n_ref, w1a_ref, w1b_ref, b1_ref, w2_ref, b2_ref,
              w3_ref, b3_ref, o_ref):
    dot = functools.partial(jnp.dot, preferred_element_type=jnp.float32,
                            precision=lax.Precision.HIGHEST)
    h = dot(p_ref[...], w1a_ref[...]) + dot(n_ref[...], w1b_ref[...])
    h = jnp.maximum(h + b1_ref[...], 0.0)
    h = jnp.maximum(dot(h, w2_ref[...]) + b2_ref[...], 0.0)
    o_ref[...] = dot(h, w3_ref[...]) + b3_ref[...]


def _mlp(p, n, w1a, w1b, b1, w2, b2, w3, b3):
    grid = (B // _BM,)
    return pl.pallas_call(
        _mlp_body,
        grid=grid,
        in_specs=[
            pl.BlockSpec((_BM, H), lambda i: (i, 0)),
            pl.BlockSpec((_BM, H), lambda i: (i, 0)),
            pl.BlockSpec((H, H), lambda i: (0, 0)),
            pl.BlockSpec((H, H), lambda i: (0, 0)),
            pl.BlockSpec((1, H), lambda i: (0, 0)),
            pl.BlockSpec((H, H), lambda i: (0, 0)),
            pl.BlockSpec((1, H), lambda i: (0, 0)),
            pl.BlockSpec((H, 1), lambda i: (0, 0)),
            pl.BlockSpec((1, 1), lambda i: (0, 0)),
        ],
        out_specs=pl.BlockSpec((_BM, 1), lambda i: (i, 0)),
        out_shape=jax.ShapeDtypeStruct((B, 1), jnp.float32),
    )(p, n, w1a, w1b, b1, w2, b2, w3, b3)


def kernel(x, emb_proton, emb_neutron, W1, b1, W2, b2, W3, b3):
    idx_p = x[:, 0].astype(jnp.int32)
    idx_n = x[:, 1].astype(jnp.int32)
    p, n = _sc_gather(emb_proton, emb_neutron, idx_p, idx_n)
    return _mlp(p, n, W1[:H], W1[H:], b1.reshape(1, H), W2,
                b2.reshape(1, H), W3, b3.reshape(1, 1))
